# Initial kernel scaffold; baseline (speedup 1.0000x reference)
import jax
import jax.numpy as jnp
from jax.experimental import pallas as pl


def kernel(x, table):
    raise NotImplementedError("write your pallas kernel here")


# speedup vs baseline: 9.3118x; 9.3118x over previous
import functools
import math

import jax
import jax.numpy as jnp
from jax import lax
from jax.experimental import pallas as pl
from jax.experimental.pallas import tpu as pltpu
from jax.experimental.pallas import tpu_sc as plsc

D_MODEL = 128
SCALE = math.sqrt(float(D_MODEL))

NUM_CORES = 2
NUM_SUBCORES = 16
NW = NUM_CORES * NUM_SUBCORES
CHUNK = 128
NBUF = 4
AHEAD = 2
LANES = 16


def _body(nchunks, idx_hbm, table_hbm, out_hbm,
          idx_v, r0, r1, r2, r3, g0, g1, g2, g3, s0, s1, s2, s3):
    rows = (r0, r1, r2, r3)
    gsem = (g0, g1, g2, g3)
    ssem = (s0, s1, s2, s3)

    wid = lax.axis_index("s") * NUM_CORES + lax.axis_index("c")
    base = wid * (nchunks * CHUNK)

    pltpu.sync_copy(idx_hbm.at[wid], idx_v)

    def gather_start(j, b):
        pltpu.make_async_copy(table_hbm.at[idx_v.at[j]], rows[b], gsem[b]).start()

    def gather_wait(b):
        pltpu.make_async_copy(table_hbm.at[idx_v.at[0]], rows[b], gsem[b]).wait()

    def scatter_start(j, b):
        dst = out_hbm.at[pl.ds(base + j * CHUNK, CHUNK)]
        pltpu.make_async_copy(rows[b], dst, ssem[b]).start()

    def scatter_wait(b):
        dst = out_hbm.at[pl.ds(base, CHUNK)]
        pltpu.make_async_copy(rows[b], dst, ssem[b]).wait()

    def scale_buf(b):
        r = rows[b]

        def srow(i, carry):
            for c in range(D_MODEL // LANES):
                sl = (i, pl.ds(c * LANES, LANES))
                r[sl] = r[sl] * SCALE
            return carry

        lax.fori_loop(0, CHUNK, srow, 0)

    for j in range(AHEAD):
        gather_start(j, j)

    def outer(g, carry):
        for b in range(NBUF):
            j = g * NBUF + b
            gather_wait(b)
            scale_buf(b)
            scatter_start(j, b)
            jn = j + AHEAD
            bn = (b + AHEAD) % NBUF

            @pl.when(jnp.logical_and(jn < nchunks, jn >= NBUF))
            def _():
                scatter_wait(bn)

            @pl.when(jn < nchunks)
            def _():
                gather_start(jn, bn)
        return carry

    lax.fori_loop(0, nchunks // NBUF, outer, 0)

    for b in range(NBUF):
        scatter_wait(b)


def _make_call(nchunks):
    mesh = plsc.VectorSubcoreMesh(core_axis_name="c", subcore_axis_name="s")
    ntok = NW * nchunks * CHUNK
    return functools.partial(
        pl.kernel,
        mesh=mesh,
        out_type=jax.ShapeDtypeStruct((ntok, D_MODEL), jnp.float32),
        scratch_types=[
            pltpu.VMEM((nchunks, CHUNK), jnp.int32),
            pltpu.VMEM((CHUNK, D_MODEL), jnp.float32),
            pltpu.VMEM((CHUNK, D_MODEL), jnp.float32),
            pltpu.VMEM((CHUNK, D_MODEL), jnp.float32),
            pltpu.VMEM((CHUNK, D_MODEL), jnp.float32),
            pltpu.SemaphoreType.DMA,
            pltpu.SemaphoreType.DMA,
            pltpu.SemaphoreType.DMA,
            pltpu.SemaphoreType.DMA,
            pltpu.SemaphoreType.DMA,
            pltpu.SemaphoreType.DMA,
            pltpu.SemaphoreType.DMA,
            pltpu.SemaphoreType.DMA,
        ],
    )(functools.partial(_body, nchunks))


def kernel(x, table):
    ntok = x.size
    assert ntok % (NW * CHUNK) == 0
    nchunks = ntok // (NW * CHUNK)
    idx = x.reshape(NW, nchunks, CHUNK).astype(jnp.int32)
    out = _make_call(nchunks)(idx, table)
    return out.reshape(x.shape + (D_MODEL,))


# speedup vs baseline: 9.3477x; 1.0039x over previous
import functools
import math

import jax
import jax.numpy as jnp
from jax import lax
from jax.experimental import pallas as pl
from jax.experimental.pallas import tpu as pltpu
from jax.experimental.pallas import tpu_sc as plsc

D_MODEL = 128
SCALE = math.sqrt(float(D_MODEL))

NUM_CORES = 2
NUM_SUBCORES = 16
NW = NUM_CORES * NUM_SUBCORES
CHUNK = 128
NBUF = 5
AHEAD = 3
LANES = 16


def _body(nchunks, idx_hbm, table_hbm, out_hbm, idx_v, *bufs):
    rows = bufs[:NBUF]
    gsem = bufs[NBUF:2 * NBUF]
    ssem = bufs[2 * NBUF:]

    wid = lax.axis_index("s") * NUM_CORES + lax.axis_index("c")
    base = wid * (nchunks * CHUNK)

    pltpu.sync_copy(idx_hbm.at[wid], idx_v)

    def gather_start(j, b):
        pltpu.make_async_copy(table_hbm.at[idx_v.at[j]], rows[b], gsem[b]).start()

    def gather_wait(b):
        pltpu.make_async_copy(table_hbm.at[idx_v.at[0]], rows[b], gsem[b]).wait()

    def scatter_start(j, b):
        dst = out_hbm.at[pl.ds(base + j * CHUNK, CHUNK)]
        pltpu.make_async_copy(rows[b], dst, ssem[b]).start()

    def scatter_wait(b):
        dst = out_hbm.at[pl.ds(base, CHUNK)]
        pltpu.make_async_copy(rows[b], dst, ssem[b]).wait()

    def scale_buf(b):
        r = rows[b]

        def srow(i, carry):
            for c in range(D_MODEL // LANES):
                sl = (i, pl.ds(c * LANES, LANES))
                r[sl] = r[sl] * SCALE
            return carry

        lax.fori_loop(0, CHUNK, srow, 0)

    for j in range(AHEAD):
        gather_start(j, j)

    def outer(g, carry):
        for b in range(NBUF):
            j = g * NBUF + b
            gather_wait(b)
            scale_buf(b)
            scatter_start(j, b)
            jn = j + AHEAD
            bn = (b + AHEAD) % NBUF

            @pl.when(jnp.logical_and(jn < nchunks, jn >= NBUF))
            def _():
                scatter_wait(bn)

            @pl.when(jn < nchunks)
            def _():
                gather_start(jn, bn)
        return carry

    lax.fori_loop(0, nchunks // NBUF, outer, 0)

    for b in range(NBUF):
        scatter_wait(b)


def _make_call(nchunks):
    mesh = plsc.VectorSubcoreMesh(core_axis_name="c", subcore_axis_name="s")
    ntok = NW * nchunks * CHUNK
    return functools.partial(
        pl.kernel,
        mesh=mesh,
        out_type=jax.ShapeDtypeStruct((ntok, D_MODEL), jnp.float32),
        scratch_types=(
            [pltpu.VMEM((nchunks, CHUNK), jnp.int32)]
            + [pltpu.VMEM((CHUNK, D_MODEL), jnp.float32) for _ in range(NBUF)]
            + [pltpu.SemaphoreType.DMA for _ in range(2 * NBUF)]
        ),
    )(functools.partial(_body, nchunks))


def kernel(x, table):
    ntok = x.size
    assert ntok % (NW * CHUNK) == 0
    nchunks = ntok // (NW * CHUNK)
    idx = x.reshape(NW, nchunks, CHUNK).astype(jnp.int32)
    out = _make_call(nchunks)(idx, table)
    return out.reshape(x.shape + (D_MODEL,))
